# Initial kernel scaffold; baseline (speedup 1.0000x reference)
import jax
import jax.numpy as jnp
from jax.experimental import pallas as pl


def kernel(z, W_emb, b_emb, W_router, W1, b1, W2, b2, W_head, b_head):
    raise NotImplementedError("write your pallas kernel here")


# speedup vs baseline: 3.0202x; 3.0202x over previous
import functools

import jax
import jax.numpy as jnp
import numpy as np
from jax.experimental import pallas as pl
from jax.experimental.pallas import tpu as pltpu

BS, NUM_PATCH, N_VARS, PATCH_LEN = 64, 64, 7, 16
D_MODEL, D_FF, N_EXPERTS, TOP_K = 128, 256, 10, 2
TOKENS = BS * N_VARS * NUM_PATCH
TILE = 512


def _sincos_pos(num_patch, d_model):
    pos = np.arange(num_patch)[:, None].astype(np.float64)
    i = np.arange(d_model)[None, :].astype(np.float64)
    angle = pos / np.power(10000.0, (2.0 * (i // 2)) / d_model)
    pe = np.zeros((num_patch, d_model), dtype=np.float32)
    pe[:, 0::2] = np.sin(angle[:, 0::2])
    pe[:, 1::2] = np.cos(angle[:, 1::2])
    return pe


def _moe_block(zt_ref, pe_ref, W_emb_ref, b_emb_ref, W_router_ref,
               W1_ref, b1_ref, W2_ref, b2_ref, W_head_ref, b_head_ref,
               y_ref):
    x = jnp.dot(zt_ref[...], W_emb_ref[...],
                preferred_element_type=jnp.float32)
    x = x + b_emb_ref[...] + pe_ref[...]

    logits = jnp.dot(x, W_router_ref[...], preferred_element_type=jnp.float32)
    probs = jax.nn.softmax(logits, axis=-1)
    e_ids = jax.lax.broadcasted_iota(jnp.int32, (TILE, N_EXPERTS), 1)
    v1 = jnp.max(probs, axis=-1, keepdims=True)
    i1 = jnp.min(jnp.where(probs == v1, e_ids, N_EXPERTS), axis=-1,
                 keepdims=True)
    m1 = e_ids == i1
    probs2 = jnp.where(m1, -1.0, probs)
    v2 = jnp.max(probs2, axis=-1, keepdims=True)
    i2 = jnp.min(jnp.where(probs2 == v2, e_ids, N_EXPERTS), axis=-1,
                 keepdims=True)
    m2 = e_ids == i2
    denom = v1 + v2
    gates = jnp.where(m1, v1 / denom, 0.0) + jnp.where(m2, v2 / denom, 0.0)

    acc = jnp.zeros((TILE, D_MODEL), dtype=jnp.float32)
    for e in range(N_EXPERTS):
        h = jnp.dot(x, W1_ref[e], preferred_element_type=jnp.float32)
        h = jax.nn.gelu(h + b1_ref[e])
        eo = jnp.dot(h, W2_ref[e], preferred_element_type=jnp.float32)
        eo = eo + b2_ref[e]
        acc = acc + gates[:, e:e + 1] * eo

    tokens = x + acc
    y_ref[...] = jnp.dot(tokens, W_head_ref[...],
                         preferred_element_type=jnp.float32) + b_head_ref[...]


@jax.jit
def kernel(z, W_emb, b_emb, W_router, W1, b1, W2, b2, W_head, b_head):
    bs, num_patch, n_vars, patch_len = z.shape
    d_model = W_emb.shape[1]
    zt = z.transpose(0, 2, 1, 3).reshape(TOKENS, patch_len)
    pe = _sincos_pos(num_patch, d_model)
    pe_tile = jnp.asarray(np.tile(pe, (TILE // num_patch, 1)))

    grid = (TOKENS // TILE,)
    y_flat = pl.pallas_call(
        _moe_block,
        grid=grid,
        in_specs=[
            pl.BlockSpec((TILE, patch_len), lambda i: (i, 0)),
            pl.BlockSpec((TILE, d_model), lambda i: (0, 0)),
            pl.BlockSpec((patch_len, d_model), lambda i: (0, 0)),
            pl.BlockSpec((d_model,), lambda i: (0,)),
            pl.BlockSpec((d_model, N_EXPERTS), lambda i: (0, 0)),
            pl.BlockSpec((N_EXPERTS, d_model, D_FF), lambda i: (0, 0, 0)),
            pl.BlockSpec((N_EXPERTS, D_FF), lambda i: (0, 0)),
            pl.BlockSpec((N_EXPERTS, D_FF, d_model), lambda i: (0, 0, 0)),
            pl.BlockSpec((N_EXPERTS, d_model), lambda i: (0, 0)),
            pl.BlockSpec((d_model, patch_len), lambda i: (0, 0)),
            pl.BlockSpec((patch_len,), lambda i: (0,)),
        ],
        out_specs=pl.BlockSpec((TILE, patch_len), lambda i: (i, 0)),
        out_shape=jax.ShapeDtypeStruct((TOKENS, patch_len), jnp.float32),
        compiler_params=pltpu.CompilerParams(
            dimension_semantics=("arbitrary",)),
    )(zt, pe_tile, W_emb, b_emb, W_router, W1, b1, W2, b2, W_head, b_head)

    y = y_flat.reshape(bs, n_vars, num_patch, patch_len)
    y = y.transpose(0, 2, 3, 1).reshape(bs, num_patch * patch_len, n_vars)
    return y
